# baseline (speedup 1.0000x reference)
import functools

import jax
import jax.numpy as jnp
from jax import lax
from jax.experimental import pallas as pl
from jax.experimental.pallas import tpu as pltpu
from jax.experimental.pallas import tpu_sc as plsc

B_USER = 16384
B_ITEM = 4096
K = 32
W = 128
R = W // K

_INFO = plsc.get_sparse_core_info()
_NC = _INFO.num_cores
_NS = _INFO.num_subcores
_NW = _NC * _NS
_CHUNK = 128

_BU_PER = B_USER // _NW
_BI_PER = B_ITEM // _NW
_NU_CH = _BU_PER // _CHUNK
_NI_CH = _BI_PER // _CHUNK


def _sc_gather_body(user_hbm, item_hbm, emb_user_hbm, emb_item_hbm,
                    u_out, v_out, uidx_v, iidx_v, urows_v, irows_v, sem):
    wid = lax.axis_index("s") * _NC + lax.axis_index("c")
    ubase = wid * _BU_PER
    ibase = wid * _BI_PER
    pltpu.sync_copy(user_hbm.at[wid], uidx_v)
    pltpu.sync_copy(item_hbm.at[wid], iidx_v)
    copies = []
    for j in range(_NU_CH):
        copies.append(pltpu.async_copy(
            emb_user_hbm.at[uidx_v.at[j]],
            urows_v.at[pl.ds(j * _CHUNK, _CHUNK)], sem))
    for j in range(_NI_CH):
        copies.append(pltpu.async_copy(
            emb_item_hbm.at[iidx_v.at[j]],
            irows_v.at[pl.ds(j * _CHUNK, _CHUNK)], sem))
    for c in copies:
        c.wait()
    pltpu.sync_copy(urows_v, u_out.at[pl.ds(ubase, _BU_PER)])
    pltpu.sync_copy(irows_v, v_out.at[pl.ds(ibase, _BI_PER)])


def _sc_gather(user_q, item_q, emb_user_w, emb_item_w):
    mesh = plsc.VectorSubcoreMesh(core_axis_name="c", subcore_axis_name="s")
    f = functools.partial(
        pl.kernel,
        mesh=mesh,
        out_type=[
            jax.ShapeDtypeStruct((B_USER, W), jnp.float32),
            jax.ShapeDtypeStruct((B_ITEM, W), jnp.float32),
        ],
        scratch_types=[
            pltpu.VMEM((_NU_CH, _CHUNK), jnp.int32),
            pltpu.VMEM((_NI_CH, _CHUNK), jnp.int32),
            pltpu.VMEM((_BU_PER, W), jnp.float32),
            pltpu.VMEM((_BI_PER, W), jnp.float32),
            pltpu.SemaphoreType.DMA,
        ],
    )(_sc_gather_body)
    return f(user_q, item_q, emb_user_w, emb_item_w)


_BM = 512


def _select32(wide, rem):
    acc = jnp.where(rem == 0, wide[:, 0:K], 0.0)
    for r in range(1, R):
        acc = acc + jnp.where(rem == r, wide[:, r * K:(r + 1) * K], 0.0)
    return acc


def _mm_body(ur_ref, vr_ref, uw_ref, vw_ref, o_ref):
    u = _select32(uw_ref[...], ur_ref[...])
    v = _select32(vw_ref[...], vr_ref[...])
    o_ref[...] = lax.dot_general(
        u, v,
        dimension_numbers=(((1,), (1,)), ((), ())),
        preferred_element_type=jnp.float32)


def _tc_matmul(user_r, item_r, uw, vw):
    return pl.pallas_call(
        _mm_body,
        grid=(B_USER // _BM,),
        in_specs=[
            pl.BlockSpec((_BM, 1), lambda i: (i, 0)),
            pl.BlockSpec((B_ITEM, 1), lambda i: (0, 0)),
            pl.BlockSpec((_BM, W), lambda i: (i, 0)),
            pl.BlockSpec((B_ITEM, W), lambda i: (0, 0)),
        ],
        out_specs=pl.BlockSpec((_BM, B_ITEM), lambda i: (i, 0)),
        out_shape=jax.ShapeDtypeStruct((B_USER, B_ITEM), jnp.float32),
    )(user_r, item_r, uw, vw)


def kernel(user, item, emb_user, emb_item):
    user = user.astype(jnp.int32)
    item = item.astype(jnp.int32)
    user_q = (user // R).reshape(_NW, _NU_CH, _CHUNK)
    item_q = (item // R).reshape(_NW, _NI_CH, _CHUNK)
    user_r = (user % R).reshape(B_USER, 1)
    item_r = (item % R).reshape(B_ITEM, 1)
    emb_user_w = emb_user.reshape(emb_user.shape[0] // R, W)
    emb_item_w = emb_item.reshape(emb_item.shape[0] // R, W)
    uw, vw = _sc_gather(user_q, item_q, emb_user_w, emb_item_w)
    return _tc_matmul(user_r, item_r, uw, vw)


# speedup vs baseline: 6.6117x; 6.6117x over previous
import jax
import jax.numpy as jnp
from jax import lax
from jax.experimental import pallas as pl

B_USER = 16384
B_ITEM = 4096
K = 32

_BM = 512


def _mm_body(u_ref, v_ref, o_ref):
    o_ref[...] = lax.dot_general(
        u_ref[...], v_ref[...],
        dimension_numbers=(((1,), (1,)), ((), ())),
        preferred_element_type=jnp.float32)


def _tc_matmul(u, v):
    return pl.pallas_call(
        _mm_body,
        grid=(B_USER // _BM,),
        in_specs=[
            pl.BlockSpec((_BM, K), lambda i: (i, 0)),
            pl.BlockSpec((B_ITEM, K), lambda i: (0, 0)),
        ],
        out_specs=pl.BlockSpec((_BM, B_ITEM), lambda i: (i, 0)),
        out_shape=jax.ShapeDtypeStruct((B_USER, B_ITEM), jnp.float32),
    )(u, v)


def kernel(user, item, emb_user, emb_item):
    u = jnp.take(emb_user, user, axis=0)
    v = jnp.take(emb_item, item, axis=0)
    return _tc_matmul(u, v)
